# Initial kernel scaffold; baseline (speedup 1.0000x reference)
import jax
import jax.numpy as jnp
from jax.experimental import pallas as pl


def kernel(x, value_emb, row_emb, col_emb, degree_emb):
    raise NotImplementedError("write your pallas kernel here")


# speedup vs baseline: 5.4494x; 5.4494x over previous
import jax
import jax.numpy as jnp
from jax import lax
from jax.experimental import pallas as pl

P = 127
MAX_DEGREE = 8
M = 16
D_MODEL = 128
DEPTH = 8
TOK = DEPTH * M * M
BB = 8


def _body(x_ref, vt_ref, row_ref, col_ref, deg_ref, out_ref):
    n = BB * TOK
    idx = x_ref[...]
    iot = lax.broadcasted_iota(jnp.int32, (n, P + 1), 1)
    onehot = (idx == iot).astype(jnp.bfloat16)
    vals = jnp.dot(onehot, vt_ref[...], preferred_element_type=jnp.float32)
    pos = (deg_ref[...][:, None, None, :]
           + row_ref[...][None, :, None, :]
           + col_ref[...][None, None, :, :]).reshape(TOK, D_MODEL)
    out_ref[...] = vals.reshape(BB, TOK, D_MODEL) + pos[None, :, :]


def kernel(x, value_emb, row_emb, col_emb, degree_emb):
    batch = x.shape[0]
    x2 = x.reshape(batch * TOK, 1)
    vt = jnp.pad(value_emb.astype(jnp.bfloat16), ((0, 1), (0, 0)))
    grid = (batch // BB,)
    out = pl.pallas_call(
        _body,
        grid=grid,
        in_specs=[
            pl.BlockSpec((BB * TOK, 1), lambda i: (i, 0)),
            pl.BlockSpec((P + 1, D_MODEL), lambda i: (0, 0)),
            pl.BlockSpec((M, D_MODEL), lambda i: (0, 0)),
            pl.BlockSpec((M, D_MODEL), lambda i: (0, 0)),
            pl.BlockSpec((MAX_DEGREE, D_MODEL), lambda i: (0, 0)),
        ],
        out_specs=pl.BlockSpec((BB, TOK, D_MODEL), lambda i: (i, 0, 0)),
        out_shape=jax.ShapeDtypeStruct((batch, TOK, D_MODEL), jnp.float32),
    )(x2, vt, row_emb, col_emb, degree_emb)
    return out.reshape(batch, DEPTH, M * M, D_MODEL)


# speedup vs baseline: 17.0821x; 3.1347x over previous
import jax
import jax.numpy as jnp
from jax import lax
from jax.experimental import pallas as pl

P = 127
MAX_DEGREE = 8
M = 16
D_MODEL = 128
DEPTH = 8
TOK = DEPTH * M * M
BB = 8


def _body(x_ref, vt_ref, row_ref, col_ref, deg_ref, out_ref):
    n = BB * TOK
    idx = jnp.broadcast_to(x_ref[0], (P + 1, n))
    iot = lax.broadcasted_iota(jnp.int32, (P + 1, n), 0)
    onehot_t = (idx == iot).astype(jnp.bfloat16)
    vals = lax.dot_general(onehot_t, vt_ref[...],
                           (((0,), (0,)), ((), ())),
                           preferred_element_type=jnp.float32)
    pos = (deg_ref[...][:, None, None, :]
           + row_ref[...][None, :, None, :]
           + col_ref[...][None, None, :, :]).reshape(TOK, D_MODEL)
    out_ref[...] = vals.reshape(BB, TOK, D_MODEL) + pos[None, :, :]


def kernel(x, value_emb, row_emb, col_emb, degree_emb):
    batch = x.shape[0]
    x2 = x.reshape(batch // BB, 1, BB * TOK)
    vt = jnp.pad(value_emb.astype(jnp.bfloat16), ((0, 1), (0, 0)))
    grid = (batch // BB,)
    out = pl.pallas_call(
        _body,
        grid=grid,
        in_specs=[
            pl.BlockSpec((1, 1, BB * TOK), lambda i: (i, 0, 0)),
            pl.BlockSpec((P + 1, D_MODEL), lambda i: (0, 0)),
            pl.BlockSpec((M, D_MODEL), lambda i: (0, 0)),
            pl.BlockSpec((M, D_MODEL), lambda i: (0, 0)),
            pl.BlockSpec((MAX_DEGREE, D_MODEL), lambda i: (0, 0)),
        ],
        out_specs=pl.BlockSpec((BB, TOK, D_MODEL), lambda i: (i, 0, 0)),
        out_shape=jax.ShapeDtypeStruct((batch, TOK, D_MODEL), jnp.float32),
    )(x2, vt, row_emb, col_emb, degree_emb)
    return out.reshape(batch, DEPTH, M * M, D_MODEL)
